# baseline (speedup 1.0000x reference)
import functools

import jax
import jax.numpy as jnp
from jax import lax
from jax.experimental import pallas as pl
from jax.experimental.pallas import tpu as pltpu
from jax.experimental.pallas import tpu_sc as plsc

_CHUNK_ROWS = 8
_NSLOT = 4


def _make_sc_add(B, S, D, NC, NS, L):
    NW = NC * NS
    rows_per_tile = S // NW
    chunk_rows = _CHUNK_ROWS
    n_chunks = rows_per_tile // chunk_rows
    NSLOT = _NSLOT

    mesh = plsc.VectorSubcoreMesh(core_axis_name="c", subcore_axis_name="s")

    scratch = (
        [pltpu.VMEM((B, chunk_rows, D), jnp.float32)] * NSLOT
        + [pltpu.VMEM((chunk_rows, D), jnp.float32)] * NSLOT
        + [pltpu.SemaphoreType.DMA] * (2 * NSLOT)
    )

    @functools.partial(
        pl.kernel,
        mesh=mesh,
        out_type=jax.ShapeDtypeStruct((B, S, D), jnp.float32),
        scratch_types=scratch,
    )
    def sc_add(x_hbm, emb_hbm, out_hbm, *refs):
        xbuf = list(refs[:NSLOT])
        ebuf = list(refs[NSLOT:2 * NSLOT])
        insem = list(refs[2 * NSLOT:3 * NSLOT])
        outsem = list(refs[3 * NSLOT:4 * NSLOT])
        wid = lax.axis_index("s") * NC + lax.axis_index("c")
        base = wid * rows_per_tile

        def in_copies(c, s):
            r0 = base + c * chunk_rows
            return [
                pltpu.make_async_copy(
                    x_hbm.at[:, pl.ds(r0, chunk_rows), :], xbuf[s], insem[s]
                ),
                pltpu.make_async_copy(
                    emb_hbm.at[pl.ds(r0, chunk_rows)], ebuf[s], insem[s]
                ),
            ]

        def out_copy(c, s):
            r0 = base + c * chunk_rows
            return pltpu.make_async_copy(
                xbuf[s], out_hbm.at[:, pl.ds(r0, chunk_rows), :], outsem[s]
            )

        def compute(s):
            xb = xbuf[s]
            eb = ebuf[s]

            @plsc.parallel_loop(0, chunk_rows, 1)
            def _(r):
                @plsc.parallel_loop(0, D, L, unroll=8)
                def _(k):
                    e = eb[r, pl.ds(k, L)]
                    for b in range(B):
                        xb[b, r, pl.ds(k, L)] = xb[b, r, pl.ds(k, L)] + e

        for s in range(NSLOT):
            for cp in in_copies(s, s):
                cp.start()

        @pl.loop(0, n_chunks - NSLOT, step=NSLOT)
        def _(g):
            for s in range(NSLOT):
                c = g + s
                for cp in in_copies(c, s):
                    cp.wait()
                compute(s)
                out_copy(c, s).start()
            for s in range(NSLOT):
                c = g + s
                out_copy(c, s).wait()
                for cp in in_copies(c + NSLOT, s):
                    cp.start()

        c0 = n_chunks - NSLOT
        for s in range(NSLOT):
            for cp in in_copies(c0 + s, s):
                cp.wait()
            compute(s)
            out_copy(c0 + s, s).start()
        for s in range(NSLOT):
            out_copy(c0 + s, s).wait()

    return sc_add


def kernel(x, emb):
    B, S, D = x.shape
    info = plsc.get_sparse_core_info()
    NC, NS, L = info.num_cores, info.num_subcores, info.num_lanes
    sc_add = _make_sc_add(B, S, D, NC, NS, L)
    return sc_add(x, emb[:S])


# speedup vs baseline: 1.0623x; 1.0623x over previous
import functools

import jax
import jax.numpy as jnp
from jax import lax
from jax.experimental import pallas as pl
from jax.experimental.pallas import tpu as pltpu
from jax.experimental.pallas import tpu_sc as plsc

_CHUNK_ROWS = 16
_NSLOT = 2
_LOOKAHEAD = 1


def _make_sc_add(B, S, D, NC, NS, L):
    NW = NC * NS
    rows_per_tile = S // NW
    chunk_rows = _CHUNK_ROWS
    n_chunks = rows_per_tile // chunk_rows
    NSLOT = _NSLOT
    LA = _LOOKAHEAD

    mesh = plsc.VectorSubcoreMesh(core_axis_name="c", subcore_axis_name="s")

    scratch = (
        [pltpu.VMEM((B, chunk_rows, D), jnp.float32)] * NSLOT
        + [pltpu.VMEM((chunk_rows, D), jnp.float32)] * NSLOT
        + [pltpu.SemaphoreType.DMA] * (2 * NSLOT)
    )

    @functools.partial(
        pl.kernel,
        mesh=mesh,
        out_type=jax.ShapeDtypeStruct((B, S, D), jnp.float32),
        scratch_types=scratch,
    )
    def sc_add(x_hbm, emb_hbm, out_hbm, *refs):
        xbuf = list(refs[:NSLOT])
        ebuf = list(refs[NSLOT:2 * NSLOT])
        insem = list(refs[2 * NSLOT:3 * NSLOT])
        outsem = list(refs[3 * NSLOT:4 * NSLOT])
        wid = lax.axis_index("s") * NC + lax.axis_index("c")
        base = wid * rows_per_tile

        def in_copies(c):
            slot = c % NSLOT
            r0 = base + c * chunk_rows
            return [
                pltpu.make_async_copy(
                    x_hbm.at[:, pl.ds(r0, chunk_rows), :],
                    xbuf[slot],
                    insem[slot],
                ),
                pltpu.make_async_copy(
                    emb_hbm.at[pl.ds(r0, chunk_rows)],
                    ebuf[slot],
                    insem[slot],
                ),
            ]

        def out_copy(c):
            slot = c % NSLOT
            r0 = base + c * chunk_rows
            return pltpu.make_async_copy(
                xbuf[slot],
                out_hbm.at[:, pl.ds(r0, chunk_rows), :],
                outsem[slot],
            )

        for c in range(min(LA, n_chunks)):
            for cp in in_copies(c):
                cp.start()
        for c in range(n_chunks):
            slot = c % NSLOT
            j = c + LA
            if j < n_chunks:
                if j >= NSLOT:
                    out_copy(j - NSLOT).wait()
                for cp in in_copies(j):
                    cp.start()
            for cp in in_copies(c):
                cp.wait()
            xb = xbuf[slot]
            eb = ebuf[slot]

            @plsc.parallel_loop(0, chunk_rows, 1)
            def _(r):
                @plsc.parallel_loop(0, D, L, unroll=8)
                def _(k):
                    e = eb[r, pl.ds(k, L)]
                    for b in range(B):
                        xb[b, r, pl.ds(k, L)] = xb[b, r, pl.ds(k, L)] + e

            out_copy(c).start()
        for c in range(max(0, n_chunks - NSLOT), n_chunks):
            out_copy(c).wait()

    return sc_add


def kernel(x, emb):
    B, S, D = x.shape
    info = plsc.get_sparse_core_info()
    NC, NS, L = info.num_cores, info.num_subcores, info.num_lanes
    sc_add = _make_sc_add(B, S, D, NC, NS, L)
    return sc_add(x, emb[:S])
